# baseline (speedup 1.0000x reference)
import functools

import jax
import jax.numpy as jnp
from jax.experimental import pallas as pl
from jax.experimental.pallas import tpu as pltpu

B, L, C, T = 128, 50, 3, 300
E = 8
K = C * T
D = 768
LP = 56


def _moe_kernel(logits_ref, masks_ref, x_ref, w_ref, b_ref, out_ref):
    bs = out_ref.shape[0]
    row0 = pl.program_id(0) * bs
    logits = logits_ref[pl.ds(row0, bs), :]
    mask = (masks_ref[pl.ds(row0, bs), :] == 1).astype(jnp.float32)
    m = jnp.max(logits, axis=1, keepdims=True)
    ex = jnp.exp(logits - m)
    gates = ex / jnp.sum(ex, axis=1, keepdims=True)
    gates = gates * mask
    gates = gates / (jnp.sum(gates, axis=1, keepdims=True) + 1e-9)

    bias = jnp.dot(gates, b_ref[...], preferred_element_type=jnp.float32)

    xblk = x_ref[...]
    accs = [None] * bs
    for e in range(E):
        y = jnp.dot(xblk, w_ref[e], preferred_element_type=jnp.float32)
        for i in range(bs):
            term = gates[i:i + 1, e:e + 1] * y[i * LP:i * LP + L, :]
            accs[i] = term if e == 0 else accs[i] + term
    for i in range(bs):
        out_ref[i] = (accs[i] + bias[i][None, :]).astype(jnp.bfloat16)


@functools.partial(jax.jit, static_argnames=("bs",))
def _run(x, logits, moe_masks, expert_W, expert_b, bs=8):
    grid = (B // bs,)
    xp = jnp.pad(x.reshape(B, L, K), ((0, 0), (0, LP - L), (0, 0)))
    xbf = xp.reshape(B * LP, K).astype(jnp.bfloat16)
    wbf = expert_W.astype(jnp.bfloat16)
    out = pl.pallas_call(
        _moe_kernel,
        grid=grid,
        in_specs=[
            pl.BlockSpec((B, E), lambda i: (0, 0)),
            pl.BlockSpec((B, E), lambda i: (0, 0)),
            pl.BlockSpec((bs * LP, K), lambda i: (i, 0)),
            pl.BlockSpec((E, K, D), lambda i: (0, 0, 0)),
            pl.BlockSpec((E, D), lambda i: (0, 0)),
        ],
        out_specs=pl.BlockSpec((bs, L, D), lambda i: (i, 0, 0)),
        out_shape=jax.ShapeDtypeStruct((B, L, D), jnp.bfloat16),
    )(logits, moe_masks, xbf, wbf, expert_b)
    return out


def kernel(cycle_curve_data, logits, moe_masks, expert_W, expert_b):
    out = _run(cycle_curve_data, logits, moe_masks.astype(jnp.int32),
               expert_W, expert_b)
    return (out, jnp.float32(0.0))


# speedup vs baseline: 1.0286x; 1.0286x over previous
import functools

import jax
import jax.numpy as jnp
from jax.experimental import pallas as pl
from jax.experimental.pallas import tpu as pltpu

B, L, C, T = 128, 50, 3, 300
E = 8
K = C * T
D = 768
LP = 56


def _moe_kernel(logits_ref, masks_ref, x_ref, w_ref, b_ref, out_ref):
    bs = out_ref.shape[0]
    row0 = pl.program_id(0) * bs
    logits = logits_ref[pl.ds(row0, bs), :]
    mask = (masks_ref[pl.ds(row0, bs), :] == 1).astype(jnp.float32)
    m = jnp.max(logits, axis=1, keepdims=True)
    ex = jnp.exp(logits - m)
    gates = ex / jnp.sum(ex, axis=1, keepdims=True)
    gates = gates * mask
    gates = gates / (jnp.sum(gates, axis=1, keepdims=True) + 1e-9)

    bias = jnp.dot(gates, b_ref[...], preferred_element_type=jnp.float32)

    xblk = x_ref[...]
    accs = [None] * bs
    for e in range(E):
        y = jnp.dot(xblk, w_ref[e], preferred_element_type=jnp.float32)
        for i in range(bs):
            term = gates[i:i + 1, e:e + 1] * y[i * L:(i + 1) * L, :]
            accs[i] = term if e == 0 else accs[i] + term
    for i in range(bs):
        out_ref[i] = (accs[i] + bias[i][None, :]).astype(jnp.bfloat16)


@functools.partial(jax.jit, static_argnames=("bs",))
def _run(x, logits, moe_masks, expert_W, expert_b, bs=8):
    grid = (B // bs,)
    xbf = x.reshape(B * L, K).astype(jnp.bfloat16)
    wbf = expert_W.astype(jnp.bfloat16)
    out = pl.pallas_call(
        _moe_kernel,
        grid=grid,
        in_specs=[
            pl.BlockSpec((B, E), lambda i: (0, 0)),
            pl.BlockSpec((B, E), lambda i: (0, 0)),
            pl.BlockSpec((bs * L, K), lambda i: (i, 0)),
            pl.BlockSpec((E, K, D), lambda i: (0, 0, 0)),
            pl.BlockSpec((E, D), lambda i: (0, 0)),
        ],
        out_specs=pl.BlockSpec((bs, L, D), lambda i: (i, 0, 0)),
        out_shape=jax.ShapeDtypeStruct((B, L, D), jnp.bfloat16),
    )(logits, moe_masks, xbf, wbf, expert_b)
    return out


def kernel(cycle_curve_data, logits, moe_masks, expert_W, expert_b):
    out = _run(cycle_curve_data, logits, moe_masks.astype(jnp.int32),
               expert_W, expert_b)
    return (out, jnp.float32(0.0))


# speedup vs baseline: 1.2878x; 1.2520x over previous
import functools

import jax
import jax.numpy as jnp
from jax.experimental import pallas as pl
from jax.experimental.pallas import tpu as pltpu

B, L, C, T = 128, 50, 3, 300
E = 8
K = C * T
D = 768
LP = 56


def _moe_kernel(logits_ref, masks_ref, x_ref, w_ref, b_ref, out_ref):
    bs = out_ref.shape[0]
    row0 = pl.program_id(0) * bs
    logits = logits_ref[pl.ds(row0, bs), :]
    mask = (masks_ref[pl.ds(row0, bs), :] == 1).astype(jnp.float32)
    m = jnp.max(logits, axis=1, keepdims=True)
    ex = jnp.exp(logits - m)
    gates = ex / jnp.sum(ex, axis=1, keepdims=True)
    gates = gates * mask
    gates = gates / (jnp.sum(gates, axis=1, keepdims=True) + 1e-9)

    bias = jnp.dot(gates, b_ref[...], preferred_element_type=jnp.float32)

    xblk = x_ref[...].reshape(bs * L, K)
    accs = [None] * bs
    for e in range(E):
        y = jnp.dot(xblk, w_ref[e], preferred_element_type=jnp.float32)
        for i in range(bs):
            term = gates[i:i + 1, e:e + 1] * y[i * L:(i + 1) * L, :]
            accs[i] = term if e == 0 else accs[i] + term
    for i in range(bs):
        out_ref[i] = (accs[i] + bias[i][None, :]).astype(jnp.bfloat16)


@functools.partial(jax.jit, static_argnames=("bs",))
def _run(x, logits, moe_masks, expert_W, expert_b, bs=8):
    grid = (B // bs,)
    xbf = x.reshape(B, L, K).astype(jnp.bfloat16)
    wbf = expert_W.astype(jnp.bfloat16)
    out = pl.pallas_call(
        _moe_kernel,
        grid=grid,
        in_specs=[
            pl.BlockSpec((B, E), lambda i: (0, 0)),
            pl.BlockSpec((B, E), lambda i: (0, 0)),
            pl.BlockSpec((bs, L, K), lambda i: (i, 0, 0)),
            pl.BlockSpec((E, K, D), lambda i: (0, 0, 0)),
            pl.BlockSpec((E, D), lambda i: (0, 0)),
        ],
        out_specs=pl.BlockSpec((bs, L, D), lambda i: (i, 0, 0)),
        out_shape=jax.ShapeDtypeStruct((B, L, D), jnp.bfloat16),
    )(logits, moe_masks, xbf, wbf, expert_b)
    return out


def kernel(cycle_curve_data, logits, moe_masks, expert_W, expert_b):
    out = _run(cycle_curve_data, logits, moe_masks.astype(jnp.int32),
               expert_W, expert_b)
    return (out, jnp.float32(0.0))


# speedup vs baseline: 1.3455x; 1.0448x over previous
import functools

import jax
import jax.numpy as jnp
from jax.experimental import pallas as pl
from jax.experimental.pallas import tpu as pltpu

B, L, C, T = 128, 50, 3, 300
E = 8
K = C * T
D = 768
LP = 56


def _moe_kernel(logits_ref, masks_ref, x_ref, w_ref, b_ref, out_ref):
    bs = out_ref.shape[0]
    row0 = pl.program_id(0) * bs
    logits = logits_ref[pl.ds(row0, bs), :]
    mask = (masks_ref[pl.ds(row0, bs), :] == 1).astype(jnp.float32)
    m = jnp.max(logits, axis=1, keepdims=True)
    ex = jnp.exp(logits - m)
    gates = ex / jnp.sum(ex, axis=1, keepdims=True)
    gates = gates * mask
    gates = gates / (jnp.sum(gates, axis=1, keepdims=True) + 1e-9)

    bias = jnp.dot(gates, b_ref[...], preferred_element_type=jnp.float32)

    xblk = x_ref[...].reshape(bs * L, K).astype(jnp.bfloat16)
    accs = [None] * bs
    for e in range(E):
        y = jnp.dot(xblk, w_ref[e], preferred_element_type=jnp.float32)
        for i in range(bs):
            term = gates[i:i + 1, e:e + 1] * y[i * L:(i + 1) * L, :]
            accs[i] = term if e == 0 else accs[i] + term
    for i in range(bs):
        out_ref[i] = (accs[i] + bias[i][None, :]).astype(jnp.bfloat16)


@functools.partial(jax.jit, static_argnames=("bs",))
def _run(x, logits, moe_masks, expert_W, expert_b, bs=16):
    grid = (B // bs,)
    xbf = x.reshape(B, L, K)
    wbf = expert_W.astype(jnp.bfloat16)
    out = pl.pallas_call(
        _moe_kernel,
        grid=grid,
        in_specs=[
            pl.BlockSpec((B, E), lambda i: (0, 0)),
            pl.BlockSpec((B, E), lambda i: (0, 0)),
            pl.BlockSpec((bs, L, K), lambda i: (i, 0, 0)),
            pl.BlockSpec((E, K, D), lambda i: (0, 0, 0)),
            pl.BlockSpec((E, D), lambda i: (0, 0)),
        ],
        out_specs=pl.BlockSpec((bs, L, D), lambda i: (i, 0, 0)),
        out_shape=jax.ShapeDtypeStruct((B, L, D), jnp.bfloat16),
    )(logits, moe_masks, xbf, wbf, expert_b)
    return out


def kernel(cycle_curve_data, logits, moe_masks, expert_W, expert_b):
    out = _run(cycle_curve_data, logits, moe_masks.astype(jnp.int32),
               expert_W, expert_b)
    return (out, jnp.float32(0.0))
